# Initial kernel scaffold; baseline (speedup 1.0000x reference)
import jax
import jax.numpy as jnp
from jax.experimental import pallas as pl


def kernel(x, conv_w, gate_W, gate_b, mlp_up_W, mlp_up_b, mlp_down_W, mlp_down_b, state_in_W, state_out_W, state_decay_logit, read_bits_W, write_bits_W, mem_value_W, mem_out_W, write_gate_W, write_gate_b, gate_long_W, gate_long_b, gate_mem_W, gate_mem_b):
    raise NotImplementedError("write your pallas kernel here")


# speedup vs baseline: 8.0738x; 8.0738x over previous
import functools

import jax
import jax.numpy as jnp
from jax.experimental import pallas as pl
from jax.experimental.pallas import tpu as pltpu

BT = 256
CK = 4
BITS = 16
H = 2


def _dg(a, b, ca, cb, prec=None):
    return jax.lax.dot_general(
        a, b, dimension_numbers=(((ca,), (cb,)), ((), ())),
        preferred_element_type=jnp.float32, precision=prec)


def _rms(v):
    return v * jax.lax.rsqrt(jnp.mean(v * v, axis=-1, keepdims=True) + 1e-6)


def _main_body(K, D, HID,
               alpha_ref, lnalpha_ref,
               xc_ref, xp_ref, convT_ref, gateW_ref, gateb_ref,
               P_ref, Pb_ref, Win_ref, Wout_ref,
               upW_ref, upb_ref, downW_ref, downb_ref,
               acc_ref, wvwg_ref, bits_ref,
               carry_ref, A_ref):
    b = pl.program_id(0)
    j = pl.program_id(1)

    @pl.when(jnp.logical_and(b == 0, j == 0))
    def _():
        rows = jax.lax.broadcasted_iota(jnp.int32, (BT, BT + 8), 0
                                        ).astype(jnp.float32)
        cols = jax.lax.broadcasted_iota(jnp.int32, (BT, BT + 8), 1
                                        ).astype(jnp.float32)
        for k in range(K):
            a_k = alpha_ref[k]
            ln_k = lnalpha_ref[k]
            expo = jnp.where(cols == 0.0, rows + 1.0, rows - (cols - 1.0))
            coef = jnp.where(cols == 0.0, 1.0, 1.0 - a_k)
            valid = jnp.logical_or(cols == 0.0,
                                   jnp.logical_and(cols - 1.0 <= rows,
                                                   cols <= float(BT)))
            A_ref[k] = jnp.where(valid, coef * jnp.exp(expo * ln_k), 0.0)

    @pl.when(j == 0)
    def _():
        carry_ref[...] = jnp.zeros_like(carry_ref)

    x = xc_ref[0]
    xp = xp_ref[0]
    h = _rms(x)
    hp = jnp.where(j == 0, 0.0, _rms(xp))

    hcat = jnp.concatenate([hp[BT - (CK - 1):], h], axis=0)
    y = hcat[CK - 1:CK - 1 + BT] * convT_ref[CK - 1:CK, :]
    for c in range(CK - 1):
        y = y + hcat[c:c + BT] * convT_ref[c:c + 1, :]

    hb = h.astype(jnp.bfloat16)
    gate = jax.nn.sigmoid(_dg(hb, gateW_ref[...], 1, 1) + gateb_ref[...])
    u = y * gate

    pm = _dg(h, P_ref[...], 1, 1, prec=jax.lax.Precision.HIGHEST) + Pb_ref[...]
    rlog = pm[:, 0:H * BITS]
    wlog = pm[:, H * BITS:2 * H * BITS]
    vals = pm[:, 2 * H * BITS:2 * H * BITS + 64]
    wg = jax.nn.sigmoid(pm[:, 128:129])
    gl = jax.nn.sigmoid(pm[:, 129:130])
    gm = jax.nn.sigmoid(pm[:, 130:131])
    wv = vals * wg
    pad = jnp.zeros((BT, 128 - 64 - 3), jnp.float32)
    wvwg_ref[0] = jnp.concatenate([wv, wg, gl, gm, pad], axis=1)
    rsign = jnp.where(rlog > 0.0, 1.0, -1.0)
    wsign = jnp.where(wlog > 0.0, 1.0, -1.0)
    bits_ref[0] = jnp.concatenate([rsign, wsign], axis=1)

    ub = u.astype(jnp.bfloat16)
    zeros7 = jnp.zeros((7, D), jnp.float32)
    long_acc = None
    for k in range(K):
        zk = _dg(ub, Win_ref[k], 1, 1)
        zaug = jnp.concatenate([carry_ref[k, 0:1, :], zk, zeros7], axis=0)
        s = _dg(A_ref[k], zaug, 1, 0)
        carry_ref[k, 0:1, :] = s[BT - 1:BT, :]
        o = _dg(s.astype(jnp.bfloat16), Wout_ref[k], 1, 1)
        long_acc = o if long_acc is None else long_acc + o

    hid = jax.nn.gelu(_dg(ub, upW_ref[...], 1, 1) + upb_ref[...],
                      approximate=True)
    mlp = _dg(hid.astype(jnp.bfloat16), downW_ref[...], 1, 1) + downb_ref[...]
    acc_ref[0] = long_acc * gl + mlp


def _match_body(nK, D,
                bitsq_ref, bitsk_ref, wvwgk_ref, wvwgq_ref,
                xq_ref, accq_ref, memW_ref,
                out_ref, num_ref):
    kk = pl.program_id(2)

    @pl.when(kk == 0)
    def _():
        num_ref[...] = jnp.zeros_like(num_ref)

    rb = bitsq_ref[0][:, 0:H * BITS].astype(jnp.bfloat16)
    wb = bitsk_ref[0][:, H * BITS:2 * H * BITS].astype(jnp.bfloat16)
    wvg = wvwgk_ref[0].astype(jnp.bfloat16)
    for hh in range(H):
        rbh = rb[:, hh * BITS:(hh + 1) * BITS]
        e = None
        for h2 in range(H):
            wbh = wb[:, h2 * BITS:(h2 + 1) * BITS]
            dm = _dg(rbh, wbh, 1, 1)
            m = (dm == float(BITS)).astype(jnp.float32)
            e = m if e is None else e + m
        num_ref[hh] += _dg(e.astype(jnp.bfloat16), wvg, 1, 0)

    @pl.when(kk == nK - 1)
    def _():
        reads = None
        for hh in range(H):
            r = num_ref[hh, :, 0:64] / (num_ref[hh, :, 64:65] + 1e-6)
            reads = r if reads is None else reads + r
        reads = reads * (1.0 / H)
        mr = _dg(reads.astype(jnp.bfloat16), memW_ref[...], 1, 1)
        gm = wvwgq_ref[0][:, 66:67]
        out_ref[0] = xq_ref[0] + accq_ref[0] + mr * gm


def kernel(x, conv_w, gate_W, gate_b, mlp_up_W, mlp_up_b, mlp_down_W,
           mlp_down_b, state_in_W, state_out_W, state_decay_logit,
           read_bits_W, write_bits_W, mem_value_W, mem_out_W, write_gate_W,
           write_gate_b, gate_long_W, gate_long_b, gate_mem_W, gate_mem_b):
    Bsz, T, D = x.shape
    K = state_decay_logit.shape[0]
    HID = mlp_up_W.shape[0]
    MEMD = mem_value_W.shape[0]
    nT = T // BT

    f32 = jnp.float32
    bf = jnp.bfloat16

    convT = jnp.zeros((8, D), f32).at[0:CK, :].set(conv_w.T)
    P = jnp.zeros((256, D), f32)
    P = P.at[0:H * BITS].set(read_bits_W)
    P = P.at[H * BITS:2 * H * BITS].set(write_bits_W)
    P = P.at[2 * H * BITS:2 * H * BITS + MEMD].set(mem_value_W)
    P = P.at[128].set(write_gate_W[0])
    P = P.at[129].set(gate_long_W[0])
    P = P.at[130].set(gate_mem_W[0])
    Pb = jnp.zeros((1, 256), f32)
    Pb = Pb.at[0, 128].set(write_gate_b[0])
    Pb = Pb.at[0, 129].set(gate_long_b[0])
    Pb = Pb.at[0, 130].set(gate_mem_b[0])
    alpha = jax.nn.sigmoid(state_decay_logit)
    lnalpha = jnp.log(alpha)
    Win3 = state_in_W.reshape(K, D, D).astype(bf)
    Wout3 = state_out_W.reshape(D, K, D).transpose(1, 0, 2).astype(bf)

    whole = lambda shape: pl.BlockSpec(shape, lambda b, j: (0,) * len(shape))
    blk = lambda last: pl.BlockSpec((1, BT, last), lambda b, j: (b, j, 0))
    xprev = pl.BlockSpec((1, BT, D), lambda b, j: (b, jnp.maximum(j - 1, 0), 0))
    smem = lambda shape: pl.BlockSpec(memory_space=pltpu.SMEM)

    acc, wvwg, bits = pl.pallas_call(
        functools.partial(_main_body, K, D, HID),
        grid=(Bsz, nT),
        in_specs=[
            smem((K,)), smem((K,)),
            blk(D), xprev,
            whole((8, D)), whole((D, D)), whole((1, D)),
            whole((256, D)), whole((1, 256)),
            whole((K, D, D)), whole((K, D, D)),
            whole((HID, D)), whole((1, HID)),
            whole((D, HID)), whole((1, D)),
        ],
        out_specs=[blk(D), blk(128), blk(64)],
        out_shape=[
            jax.ShapeDtypeStruct((Bsz, T, D), f32),
            jax.ShapeDtypeStruct((Bsz, T, 128), f32),
            jax.ShapeDtypeStruct((Bsz, T, 64), f32),
        ],
        scratch_shapes=[
            pltpu.VMEM((K, 8, D), f32),
            pltpu.VMEM((K, BT, BT + 8), f32),
        ],
        compiler_params=pltpu.CompilerParams(
            dimension_semantics=("arbitrary", "arbitrary")),
    )(alpha, lnalpha,
      x, x, convT, gate_W.astype(bf), gate_b.reshape(1, D),
      P, Pb, Win3, Wout3,
      mlp_up_W.astype(bf), mlp_up_b.reshape(1, HID),
      mlp_down_W.astype(bf), mlp_down_b.reshape(1, D))

    qmap = lambda b, q, k: (b, q, 0)
    kmap = lambda b, q, k: (b, k, 0)
    out = pl.pallas_call(
        functools.partial(_match_body, nT, D),
        grid=(Bsz, nT, nT),
        in_specs=[
            pl.BlockSpec((1, BT, 2 * H * BITS), qmap),
            pl.BlockSpec((1, BT, 2 * H * BITS), kmap),
            pl.BlockSpec((1, BT, 128), kmap),
            pl.BlockSpec((1, BT, 128), qmap),
            pl.BlockSpec((1, BT, D), qmap),
            pl.BlockSpec((1, BT, D), qmap),
            pl.BlockSpec((D, MEMD), lambda b, q, k: (0, 0)),
        ],
        out_specs=pl.BlockSpec((1, BT, D), qmap),
        out_shape=jax.ShapeDtypeStruct((Bsz, T, D), f32),
        scratch_shapes=[pltpu.VMEM((H, BT, 128), f32)],
        compiler_params=pltpu.CompilerParams(
            dimension_semantics=("parallel", "arbitrary", "arbitrary")),
    )(bits, bits, wvwg, wvwg, x, acc, mem_out_W.astype(bf))
    return out


# speedup vs baseline: 8.1495x; 1.0094x over previous
import functools

import jax
import jax.numpy as jnp
from jax.experimental import pallas as pl
from jax.experimental.pallas import tpu as pltpu

BT = 256
CK = 4
BITS = 16
H = 2


def _dg(a, b, ca, cb, prec=None):
    return jax.lax.dot_general(
        a, b, dimension_numbers=(((ca,), (cb,)), ((), ())),
        preferred_element_type=jnp.float32, precision=prec)


def _rms(v):
    return v * jax.lax.rsqrt(jnp.mean(v * v, axis=-1, keepdims=True) + 1e-6)


def _main_body(K, D, HID,
               alpha_ref, lnalpha_ref,
               xc_ref, xp_ref, convT_ref, gateW_ref, gateb_ref,
               P_ref, Pb_ref, Win_ref, Wout_ref,
               upW_ref, upb_ref, downW_ref, downb_ref,
               acc_ref, wvwg_ref, bits_ref,
               carry_ref, A_ref):
    b = pl.program_id(0)
    j = pl.program_id(1)

    @pl.when(jnp.logical_and(b == 0, j == 0))
    def _():
        rows = jax.lax.broadcasted_iota(jnp.int32, (BT, BT + 8), 0
                                        ).astype(jnp.float32)
        cols = jax.lax.broadcasted_iota(jnp.int32, (BT, BT + 8), 1
                                        ).astype(jnp.float32)
        for k in range(K):
            a_k = alpha_ref[k]
            ln_k = lnalpha_ref[k]
            expo = jnp.where(cols == 0.0, rows + 1.0, rows - (cols - 1.0))
            coef = jnp.where(cols == 0.0, 1.0, 1.0 - a_k)
            valid = jnp.logical_or(cols == 0.0,
                                   jnp.logical_and(cols - 1.0 <= rows,
                                                   cols <= float(BT)))
            A_ref[k] = jnp.where(valid, coef * jnp.exp(expo * ln_k), 0.0)

    @pl.when(j == 0)
    def _():
        carry_ref[...] = jnp.zeros_like(carry_ref)

    x = xc_ref[0]
    xp = xp_ref[0]
    h = _rms(x)
    hp = jnp.where(j == 0, 0.0, _rms(xp))

    hcat = jnp.concatenate([hp[BT - (CK - 1):], h], axis=0)
    y = hcat[CK - 1:CK - 1 + BT] * convT_ref[CK - 1:CK, :]
    for c in range(CK - 1):
        y = y + hcat[c:c + BT] * convT_ref[c:c + 1, :]

    hb = h.astype(jnp.bfloat16)
    gate = jax.nn.sigmoid(_dg(hb, gateW_ref[...], 1, 1) + gateb_ref[...])
    u = y * gate

    pm = _dg(h, P_ref[...], 1, 1, prec=jax.lax.Precision.HIGHEST) + Pb_ref[...]
    rlog = pm[:, 0:H * BITS]
    wlog = pm[:, H * BITS:2 * H * BITS]
    vals = pm[:, 2 * H * BITS:2 * H * BITS + 64]
    wg = jax.nn.sigmoid(pm[:, 128:129])
    gl = jax.nn.sigmoid(pm[:, 129:130])
    gm = jax.nn.sigmoid(pm[:, 130:131])
    wv = vals * wg
    pad = jnp.zeros((BT, 128 - 64 - 3), jnp.float32)
    wvwg_ref[0] = jnp.concatenate([wv, wg, gl, gm, pad], axis=1)
    rsign = jnp.where(rlog > 0.0, 1.0, -1.0)
    wsign = jnp.where(wlog > 0.0, 1.0, -1.0)
    bits_ref[0] = jnp.concatenate([rsign, wsign], axis=1)

    ub = u.astype(jnp.bfloat16)
    zeros7 = jnp.zeros((7, D), jnp.float32)
    long_acc = None
    for k in range(K):
        zk = _dg(ub, Win_ref[k * D:(k + 1) * D, :], 1, 1)
        zaug = jnp.concatenate([carry_ref[k, 0:1, :], zk, zeros7], axis=0)
        s = _dg(A_ref[k], zaug, 1, 0)
        carry_ref[k, 0:1, :] = s[BT - 1:BT, :]
        o = _dg(s.astype(jnp.bfloat16), Wout_ref[:, k * D:(k + 1) * D], 1, 1)
        long_acc = o if long_acc is None else long_acc + o

    hid = jax.nn.gelu(_dg(ub, upW_ref[...], 1, 1) + upb_ref[...],
                      approximate=True)
    mlp = _dg(hid.astype(jnp.bfloat16), downW_ref[...], 1, 1) + downb_ref[...]
    acc_ref[0] = long_acc * gl + mlp


def _match_body(nK, D,
                bitsq_ref, bitsk_ref, wvwgk_ref, wvwgq_ref,
                xq_ref, accq_ref, memW_ref,
                out_ref, num_ref):
    kk = pl.program_id(2)

    @pl.when(kk == 0)
    def _():
        num_ref[...] = jnp.zeros_like(num_ref)

    rb = bitsq_ref[0][:, 0:H * BITS].astype(jnp.bfloat16)
    wb = bitsk_ref[0][:, H * BITS:2 * H * BITS].astype(jnp.bfloat16)
    wvg = wvwgk_ref[0].astype(jnp.bfloat16)
    for hh in range(H):
        rbh = rb[:, hh * BITS:(hh + 1) * BITS]
        e = None
        for h2 in range(H):
            wbh = wb[:, h2 * BITS:(h2 + 1) * BITS]
            dm = _dg(rbh, wbh, 1, 1)
            m = (dm == float(BITS)).astype(jnp.float32)
            e = m if e is None else e + m
        num_ref[hh] += _dg(e.astype(jnp.bfloat16), wvg, 1, 0)

    @pl.when(kk == nK - 1)
    def _():
        reads = None
        for hh in range(H):
            r = num_ref[hh, :, 0:64] / (num_ref[hh, :, 64:65] + 1e-6)
            reads = r if reads is None else reads + r
        reads = reads * (1.0 / H)
        mr = _dg(reads.astype(jnp.bfloat16), memW_ref[...], 1, 1)
        gm = wvwgq_ref[0][:, 66:67]
        out_ref[0] = xq_ref[0] + accq_ref[0] + mr * gm


def kernel(x, conv_w, gate_W, gate_b, mlp_up_W, mlp_up_b, mlp_down_W,
           mlp_down_b, state_in_W, state_out_W, state_decay_logit,
           read_bits_W, write_bits_W, mem_value_W, mem_out_W, write_gate_W,
           write_gate_b, gate_long_W, gate_long_b, gate_mem_W, gate_mem_b):
    Bsz, T, D = x.shape
    K = state_decay_logit.shape[0]
    HID = mlp_up_W.shape[0]
    MEMD = mem_value_W.shape[0]
    nT = T // BT

    f32 = jnp.float32
    bf = jnp.bfloat16

    convT = jnp.zeros((8, D), f32).at[0:CK, :].set(conv_w.T)
    P = jnp.zeros((256, D), f32)
    P = P.at[0:H * BITS].set(read_bits_W)
    P = P.at[H * BITS:2 * H * BITS].set(write_bits_W)
    P = P.at[2 * H * BITS:2 * H * BITS + MEMD].set(mem_value_W)
    P = P.at[128].set(write_gate_W[0])
    P = P.at[129].set(gate_long_W[0])
    P = P.at[130].set(gate_mem_W[0])
    Pb = jnp.zeros((1, 256), f32)
    Pb = Pb.at[0, 128].set(write_gate_b[0])
    Pb = Pb.at[0, 129].set(gate_long_b[0])
    Pb = Pb.at[0, 130].set(gate_mem_b[0])
    alpha = jax.nn.sigmoid(state_decay_logit)
    lnalpha = jnp.log(alpha)
    Win2 = state_in_W.astype(bf)
    Wout2 = state_out_W.astype(bf)

    whole = lambda shape: pl.BlockSpec(shape, lambda b, j: (0,) * len(shape))
    blk = lambda last: pl.BlockSpec((1, BT, last), lambda b, j: (b, j, 0))
    xprev = pl.BlockSpec((1, BT, D), lambda b, j: (b, jnp.maximum(j - 1, 0), 0))
    smem = lambda shape: pl.BlockSpec(memory_space=pltpu.SMEM)

    acc, wvwg, bits = pl.pallas_call(
        functools.partial(_main_body, K, D, HID),
        grid=(Bsz, nT),
        in_specs=[
            smem((K,)), smem((K,)),
            blk(D), xprev,
            whole((8, D)), whole((D, D)), whole((1, D)),
            whole((256, D)), whole((1, 256)),
            whole((K * D, D)), whole((D, K * D)),
            whole((HID, D)), whole((1, HID)),
            whole((D, HID)), whole((1, D)),
        ],
        out_specs=[blk(D), blk(128), blk(64)],
        out_shape=[
            jax.ShapeDtypeStruct((Bsz, T, D), f32),
            jax.ShapeDtypeStruct((Bsz, T, 128), f32),
            jax.ShapeDtypeStruct((Bsz, T, 64), f32),
        ],
        scratch_shapes=[
            pltpu.VMEM((K, 8, D), f32),
            pltpu.VMEM((K, BT, BT + 8), f32),
        ],
        compiler_params=pltpu.CompilerParams(
            dimension_semantics=("arbitrary", "arbitrary")),
    )(alpha, lnalpha,
      x, x, convT, gate_W.astype(bf), gate_b.reshape(1, D),
      P, Pb, Win2, Wout2,
      mlp_up_W.astype(bf), mlp_up_b.reshape(1, HID),
      mlp_down_W.astype(bf), mlp_down_b.reshape(1, D))

    qmap = lambda b, q, k: (b, q, 0)
    kmap = lambda b, q, k: (b, k, 0)
    out = pl.pallas_call(
        functools.partial(_match_body, nT, D),
        grid=(Bsz, nT, nT),
        in_specs=[
            pl.BlockSpec((1, BT, 2 * H * BITS), qmap),
            pl.BlockSpec((1, BT, 2 * H * BITS), kmap),
            pl.BlockSpec((1, BT, 128), kmap),
            pl.BlockSpec((1, BT, 128), qmap),
            pl.BlockSpec((1, BT, D), qmap),
            pl.BlockSpec((1, BT, D), qmap),
            pl.BlockSpec((D, MEMD), lambda b, q, k: (0, 0)),
        ],
        out_specs=pl.BlockSpec((1, BT, D), qmap),
        out_shape=jax.ShapeDtypeStruct((Bsz, T, D), f32),
        scratch_shapes=[pltpu.VMEM((H, BT, 128), f32)],
        compiler_params=pltpu.CompilerParams(
            dimension_semantics=("parallel", "arbitrary", "arbitrary")),
    )(bits, bits, wvwg, wvwg, x, acc, mem_out_W.astype(bf))
    return out
